# Initial kernel scaffold; baseline (speedup 1.0000x reference)
import jax
import jax.numpy as jnp
from jax.experimental import pallas as pl


def kernel(x):
    raise NotImplementedError("write your pallas kernel here")


# speedup vs baseline: 5.2529x; 5.2529x over previous
import functools

import numpy as np
import jax
import jax.numpy as jnp
from jax import lax
from jax.experimental import pallas as pl

_MAX_WIDTH = 64
_STRIPES_NUM = 2
_FILL = 0.0


@functools.lru_cache(maxsize=None)
def _stripes(total_width: int):
    with jax.ensure_compile_time_eval():
        mw = min(_MAX_WIDTH, total_width)
        key = jax.random.key(42)
        key, k1 = jax.random.split(key)
        widths = jax.random.randint(k1, (_STRIPES_NUM,), 0, mw)
        starts = []
        for i in range(_STRIPES_NUM):
            key, k = jax.random.split(key)
            starts.append(jax.random.randint(k, (), 0, total_width - widths[i]))
        widths = [int(w) for w in np.asarray(widths)]
        starts = [int(s) for s in np.asarray(jnp.stack(starts))]
    return tuple(widths), tuple(starts)


def _body(x_ref, o_ref, *, s0, e0, s1, e1):
    r = lax.broadcasted_iota(jnp.int32, x_ref.shape, 1)
    m = ((r >= s0) & (r < e0)) | ((r >= s1) & (r < e1))
    o_ref[...] = jnp.where(m, jnp.float32(_FILL), x_ref[...])


def kernel(x):
    B, T, F = x.shape
    (w0, w1), (s0, s1) = _stripes(T)
    BB = 4
    return pl.pallas_call(
        functools.partial(_body, s0=s0, e0=s0 + w0, s1=s1, e1=s1 + w1),
        grid=(B // BB,),
        in_specs=[pl.BlockSpec((BB, T, F), lambda i: (i, 0, 0))],
        out_specs=pl.BlockSpec((BB, T, F), lambda i: (i, 0, 0)),
        out_shape=jax.ShapeDtypeStruct(x.shape, x.dtype),
    )(x)
